# baseline (speedup 1.0000x reference)
import jax
import jax.numpy as jnp
from jax import lax
from jax.experimental import pallas as pl
from jax.experimental.pallas import tpu as pltpu
from jax.experimental.pallas import tpu_sc as plsc

_NUM_EMBED = 1000000
_EMBED_DIM = 32
_BATCH = 16384
_FIELDS = 26

_NC = 2
_NS = 16
_NW = _NC * _NS

_BPW = _BATCH // _NW
_RPW = _BPW * _FIELDS
_SEG_B = 32
_SEG_R = _SEG_B * _FIELDS
_DESC = 104
_NDESC = _SEG_R // _DESC
_NSEG = _BPW // _SEG_B
_NBUF = 2
_ROUNDS = _NSEG // _NBUF


def _gather_body(table_hbm, idx_hbm, out_hbm, idx_v, *rest):
    rows = rest[:_NBUF]
    gsem = rest[_NBUF:2 * _NBUF]
    wsem = rest[2 * _NBUF:3 * _NBUF]

    wid = lax.axis_index("s") * _NC + lax.axis_index("c")
    rbase = wid * _RPW
    bbase = wid * _BPW
    pltpu.sync_copy(idx_hbm.at[pl.ds(rbase, _RPW)], idx_v)

    def fire(seg, b):
        for k in range(_NDESC):
            pltpu.async_copy(
                table_hbm.at[idx_v.at[pl.ds(seg * _SEG_R + k * _DESC, _DESC)]],
                rows[b].at[pl.ds(k * _DESC, _DESC)], gsem[b])

    def drain_gather(b):
        pltpu.make_async_copy(
            table_hbm.at[pl.ds(0, _SEG_R)], rows[b], gsem[b]).wait()

    def write(seg, b):
        for k in range(_SEG_B):
            pltpu.async_copy(
                rows[b].at[pl.ds(k * _FIELDS, _FIELDS)],
                out_hbm.at[bbase + seg * _SEG_B + k], wsem[b])

    def drain_write(b):
        pltpu.make_async_copy(
            table_hbm.at[pl.ds(0, _SEG_R)], rows[b], wsem[b]).wait()

    for b in range(_NBUF):
        fire(b, b)

    @pl.loop(0, _ROUNDS - 1)
    def _(t):
        s0 = t * _NBUF
        for b in range(_NBUF):
            drain_gather(b)
            write(s0 + b, b)
        for b in range(_NBUF):
            drain_write(b)
            fire(s0 + _NBUF + b, b)

    s0 = (_ROUNDS - 1) * _NBUF
    for b in range(_NBUF):
        drain_gather(b)
        write(s0 + b, b)
    for b in range(_NBUF):
        drain_write(b)


@jax.jit
def _gather(table, idx):
    mesh = plsc.VectorSubcoreMesh(core_axis_name="c", subcore_axis_name="s")
    return pl.kernel(
        _gather_body,
        out_type=jax.ShapeDtypeStruct((_BATCH, _FIELDS, _EMBED_DIM),
                                      jnp.float32),
        mesh=mesh,
        scratch_types=(
            [pltpu.VMEM((_RPW,), jnp.int32)]
            + [pltpu.VMEM((_SEG_R, _EMBED_DIM), jnp.float32)] * _NBUF
            + [pltpu.SemaphoreType.DMA] * (2 * _NBUF)
        ),
        compiler_params=pltpu.CompilerParams(use_tc_tiling_on_sc=False),
    )(table, idx)


def kernel(x, table):
    return _gather(table, x.astype(jnp.int32).reshape(-1))
